# baseline (speedup 1.0000x reference)
import functools

import jax
import jax.numpy as jnp
from jax import lax
from jax.experimental import pallas as pl
from jax.experimental.pallas import tpu as pltpu
from jax.experimental.pallas import tpu_sc as plsc

_NW = 32
_TOT = 128 * 768 * 256 // 2
_PW = _TOT // _NW
_CH = 98304
_NCHK = _PW // _CH


@functools.partial(
    pl.kernel,
    out_type=jax.ShapeDtypeStruct((_NW * 768,), jnp.float32),
    mesh=plsc.VectorSubcoreMesh(core_axis_name="c", subcore_axis_name="s"),
    scratch_types=[pltpu.VMEM((_CH,), jnp.float32), pltpu.VMEM((768,), jnp.float32)],
)
def _scstream(x_hbm, o_hbm, buf, ob):
    wid = lax.axis_index("s") * 2 + lax.axis_index("c")
    base = wid * _PW
    acc = jnp.zeros((16,), jnp.float32)
    for k in range(_NCHK):
        pltpu.sync_copy(x_hbm.at[pl.ds(base + k * _CH, _CH)], buf)
        acc = acc + buf[pl.ds(0, 16)]
    ob[pl.ds(0, 16)] = acc
    pltpu.sync_copy(ob, o_hbm.at[pl.ds(wid * 768, 768)])


def kernel(x, W1, W2):
    xf = x.reshape(-1)[:_TOT]
    s = _scstream(xf)
    out = jnp.zeros((128, 768, 1, 1), jnp.float32) + s[0]
    return out


# speedup vs baseline: 2.6861x; 2.6861x over previous
import functools

import jax
import jax.numpy as jnp
from jax import lax
from jax.experimental import pallas as pl
from jax.experimental.pallas import tpu as pltpu
from jax.experimental.pallas import tpu_sc as plsc

_BB = 16
_B = 128
_C = 768
_KEEP = 384
_NW = 32
_RPW = _B // _NW
_NCH = _C // 16


def _tc_body(x_ref, w1t_ref, w2t_ref, o_ref):
    xv = x_ref[...]
    m = jnp.mean(xv, axis=2)
    h1 = jnp.maximum(jnp.dot(m, w1t_ref[...], preferred_element_type=jnp.float32), 0.0)
    z = jnp.dot(h1, w2t_ref[...], preferred_element_type=jnp.float32)
    o_ref[...] = 1.0 / (1.0 + jnp.exp(-z))


def _gate_tc(xr, W1t, W2t, row_off, nrows):
    c = xr.shape[1]
    blk_off = row_off // _BB
    return pl.pallas_call(
        _tc_body,
        grid=(nrows // _BB,),
        in_specs=[
            pl.BlockSpec((_BB, c, xr.shape[2]), lambda i: (i + blk_off, 0, 0)),
            pl.BlockSpec(W1t.shape, lambda i: (0, 0)),
            pl.BlockSpec(W2t.shape, lambda i: (0, 0)),
        ],
        out_specs=pl.BlockSpec((_BB, c), lambda i: (i, 0)),
        out_shape=jax.ShapeDtypeStruct((nrows, c), jnp.float32),
    )(xr, W1t, W2t)


_GDN = lax.GatherDimensionNumbers(
    offset_dims=(), collapsed_slice_dims=(0,), start_index_map=(0,)
)


def _shuffle(v, idx):
    return lax.gather(
        v,
        idx.reshape(16, 1),
        _GDN,
        slice_sizes=(1,),
        mode=lax.GatherScatterMode.PROMISE_IN_BOUNDS,
    )


def _lane_sum(v):
    lane = lax.iota(jnp.int32, 16)
    for sh in (1, 2, 4, 8):
        v = v + _shuffle(v, lane ^ sh)
    return v


def _count(mask_bool):
    return _lane_sum(jnp.where(mask_bool, 1, 0))


def _row_topk(buf, r):
    zero = jnp.zeros((16,), jnp.int32)
    keepn = jnp.full((16,), _KEEP, jnp.int32)

    one = jnp.ones((16,), jnp.int32)

    def count_ge(cand):
        acc = zero
        for ch in range(_NCH):
            acc = acc + jnp.where(buf[r, pl.ds(ch * 16, 16)] >= cand, one, zero)
        return _lane_sum(acc)

    def bit_step(i, t):
        cand = t | jnp.broadcast_to(jnp.left_shift(jnp.int32(1), 30 - i), (16,))
        return jnp.where(count_ge(cand) >= keepn, cand, t)

    t = lax.fori_loop(0, 31, bit_step, zero)

    accg = zero
    for ch in range(_NCH):
        accg = accg + jnp.where(buf[r, pl.ds(ch * 16, 16)] > t, one, zero)
    need = keepn - _lane_sum(accg)

    lane = lax.iota(jnp.int32, 16)

    def idx_step(i, j):
        cand = j | jnp.broadcast_to(jnp.left_shift(jnp.int32(1), 9 - i), (16,))
        acc = zero
        for ch in range(_NCH):
            v = buf[r, pl.ds(ch * 16, 16)]
            idx = lane + (ch * 16)
            acc = acc + jnp.where((v == t) & (idx >= cand), one, zero)
        return jnp.where(_lane_sum(acc) >= need, cand, j)

    j = lax.fori_loop(0, 10, idx_step, zero)

    for ch in range(_NCH):
        v = buf[r, pl.ds(ch * 16, 16)]
        idx = lane + (ch * 16)
        keep = (v > t) | ((v == t) & (idx >= j))
        buf[r, pl.ds(ch * 16, 16)] = jnp.where(keep, v, zero)


def _make_topk_sc(nrows):
    rpw = nrows // _NW

    @functools.partial(
        pl.kernel,
        out_type=jax.ShapeDtypeStruct((nrows, _C), jnp.int32),
        mesh=plsc.VectorSubcoreMesh(core_axis_name="c", subcore_axis_name="s"),
        scratch_types=[pltpu.VMEM((rpw, _C), jnp.int32)],
    )
    def _topk_sc(y_hbm, o_hbm, buf):
        wid = lax.axis_index("s") * 2 + lax.axis_index("c")
        base = wid * rpw
        pltpu.sync_copy(y_hbm.at[pl.ds(base, rpw)], buf)
        for r in range(rpw):
            _row_topk(buf, r)
        pltpu.sync_copy(buf, o_hbm.at[pl.ds(base, rpw)])

    return _topk_sc


_NSPLIT = 2
_topk_sc_part = _make_topk_sc(_B // _NSPLIT)


def kernel(x, W1, W2):
    b, c, h, w = x.shape
    xr = x.reshape(b, c, h * w)
    W1t, W2t = W1.T, W2.T
    rows = b // _NSPLIT
    parts = []
    for s in range(_NSPLIT):
        y = _gate_tc(xr, W1t, W2t, s * rows, rows)
        yi = lax.bitcast_convert_type(y, jnp.int32)
        parts.append(lax.bitcast_convert_type(_topk_sc_part(yi), jnp.float32))
    out = jnp.concatenate(parts, axis=0)
    return out.reshape(b, c, 1, 1)
